# Initial kernel scaffold; baseline (speedup 1.0000x reference)
import jax
import jax.numpy as jnp
from jax.experimental import pallas as pl


def kernel(images, features, W1, b1, W_obj, b_obj, W_box, b_box):
    raise NotImplementedError("write your pallas kernel here")


# speedup vs baseline: 13.7425x; 13.7425x over previous
import numpy as np

import jax
import jax.numpy as jnp
from jax import lax
from jax.experimental import pallas as pl
from jax.experimental.pallas import tpu as pltpu

_B = 2
_C = 128
_H = 48
_W = 48
_A = 3
_STRIDE = 16
_SIZE = 256.0
_IMG = 768
_PRE = 1000
_POST = 100
_THR = 0.7
_ROWS = _H * _W
_K = 1024
_CLIP = float(np.log(1000.0 / 16.0))


def _anchor_table():
    ratios = np.array([0.5, 1.0, 2.0], np.float32)
    ws = _SIZE * np.sqrt(1.0 / ratios)
    hs = _SIZE * np.sqrt(ratios)
    base = np.stack([-0.5 * (ws - 1), -0.5 * (hs - 1),
                     0.5 * (ws - 1), 0.5 * (hs - 1)], axis=1)
    sx = np.arange(_W, dtype=np.float32) * _STRIDE
    sy = np.arange(_H, dtype=np.float32) * _STRIDE
    gy, gx = np.meshgrid(sy, sx, indexing="ij")
    shifts = np.stack([gx, gy, gx, gy], axis=-1)
    anc = shifts[:, :, None, :] + base[None, None, :, :]
    return jnp.asarray(anc.reshape(-1, 4))


def _conv_head_kernel(x_ref, w1_ref, b1_ref, wh_ref, bh_ref, out_ref):
    acc = lax.dot_general(
        x_ref[0], w1_ref[...], (((1,), (0,)), ((), ())),
        preferred_element_type=jnp.float32)
    t = jnp.maximum(acc + b1_ref[0], 0.0)
    h = lax.dot_general(
        t, wh_ref[...], (((1,), (0,)), ((), ())),
        preferred_element_type=jnp.float32) + bh_ref[0]
    col = lax.broadcasted_iota(jnp.int32, h.shape, 1)
    out_ref[0] = jnp.where(col < _A, jax.nn.sigmoid(h), h)


def _decode_from(get):
    wa = get(2) - get(0) + 1.0
    ha = get(3) - get(1) + 1.0
    cxa = get(0) + 0.5 * wa
    cya = get(1) + 0.5 * ha
    dx, dy = get(4), get(5)
    dw = jnp.minimum(get(6), _CLIP)
    dh = jnp.minimum(get(7), _CLIP)
    cx = dx * wa + cxa
    cy = dy * ha + cya
    pw = jnp.exp(dw) * wa
    ph = jnp.exp(dh) * ha
    lim_w = float(_IMG - 1)
    lim_h = float(_IMG - 1)
    x1 = jnp.clip(cx - 0.5 * pw, 0.0, lim_w)
    y1 = jnp.clip(cy - 0.5 * ph, 0.0, lim_h)
    x2 = jnp.clip(cx + 0.5 * pw - 1.0, 0.0, lim_w)
    y2 = jnp.clip(cy + 0.5 * ph - 1.0, 0.0, lim_h)
    return x1, y1, x2, y2


def _nms_kernel(adc_ref, adt_ref, sc_ref, boxes_ref, keep_ref, iou_ref):
    adc = adc_ref[0]
    adt = adt_ref[0]
    x1c, y1c, x2c, y2c = _decode_from(lambda i: adc[:, i:i + 1])
    x1r, y1r, x2r, y2r = _decode_from(lambda i: adt[i:i + 1, :])
    boxes_ref[0] = jnp.concatenate([x1c, y1c, x2c, y2c], axis=1)

    area_c = (x2c - x1c + 1.0) * (y2c - y1c + 1.0)
    area_r = (x2r - x1r + 1.0) * (y2r - y1r + 1.0)
    iw = jnp.maximum(jnp.minimum(x2c, x2r) - jnp.maximum(x1c, x1r) + 1.0, 0.0)
    ih = jnp.maximum(jnp.minimum(y2c, y2r) - jnp.maximum(y1c, y1r) + 1.0, 0.0)
    inter = iw * ih
    iou_ref[...] = inter / (area_c + area_r - inter)

    s = sc_ref[0]
    iota = lax.broadcasted_iota(jnp.int32, (1, _K), 1)

    def body(_, carry):
        active, keep = carry
        cand = jnp.where(active > 0.5, s, -1.0)
        m = jnp.max(cand)
        idx = jnp.min(jnp.where(cand == m, iota, _K))
        valid = m > -0.5
        row = iou_ref[pl.ds(idx, 1), :]
        selmask = ((iota == idx) & valid).astype(jnp.float32)
        keep = keep + selmask
        new_active = active * (row <= _THR).astype(jnp.float32)
        active = jnp.where(valid, new_active, active)
        return active, keep

    active0 = jnp.ones((1, _K), jnp.float32)
    keep0 = jnp.zeros((1, _K), jnp.float32)
    _, keep = lax.fori_loop(0, _POST, body, (active0, keep0))
    keep_ref[0] = keep


def kernel(images, features, W1, b1, W_obj, b_obj, W_box, b_box):
    xp = jnp.pad(jnp.transpose(features, (0, 2, 3, 1)),
                 ((0, 0), (1, 1), (1, 1), (0, 0)))
    pats = jnp.stack([xp[:, dy:dy + _H, dx:dx + _W, :]
                      for dy in range(3) for dx in range(3)], axis=3)
    x9 = pats.reshape(_B, _ROWS, 9 * _C)

    w9 = jnp.transpose(W1, (2, 3, 1, 0)).reshape(9 * _C, _C)
    b1r = b1.reshape(1, _C)
    wh = jnp.concatenate([W_obj.reshape(_A, _C).T,
                          W_box.reshape(4 * _A, _C).T,
                          jnp.zeros((_C, 1), jnp.float32)], axis=1)
    bh = jnp.concatenate([b_obj, b_box,
                          jnp.zeros((1,), jnp.float32)]).reshape(1, 16)

    heads = pl.pallas_call(
        _conv_head_kernel,
        grid=(_B,),
        in_specs=[
            pl.BlockSpec((1, _ROWS, 9 * _C), lambda b: (b, 0, 0)),
            pl.BlockSpec((9 * _C, _C), lambda b: (0, 0)),
            pl.BlockSpec((1, _C), lambda b: (0, 0)),
            pl.BlockSpec((_C, 16), lambda b: (0, 0)),
            pl.BlockSpec((1, 16), lambda b: (0, 0)),
        ],
        out_specs=pl.BlockSpec((1, _ROWS, 16), lambda b: (b, 0, 0)),
        out_shape=jax.ShapeDtypeStruct((_B, _ROWS, 16), jnp.float32),
    )(x9, w9, b1r, wh, bh)

    scores = heads[..., :_A].reshape(_B, _H * _W * _A)
    deltas = heads[..., _A:_A + 4 * _A].reshape(_B, _H * _W * _A, 4)

    anchors = _anchor_table()
    top_s, top_i = lax.top_k(scores, _PRE)
    anc_t = jnp.take(anchors, top_i, axis=0)
    del_t = jnp.take_along_axis(deltas, top_i[..., None], axis=1)

    adc = jnp.concatenate([anc_t, del_t], axis=-1)
    adc = jnp.pad(adc, ((0, 0), (0, _K - _PRE), (0, 0)))
    adt = jnp.transpose(adc, (0, 2, 1))
    sc3 = jnp.pad(top_s, ((0, 0), (0, _K - _PRE)),
                  constant_values=-1.0).reshape(_B, 1, _K)

    boxes_dec, keep = pl.pallas_call(
        _nms_kernel,
        grid=(_B,),
        in_specs=[
            pl.BlockSpec((1, _K, 8), lambda b: (b, 0, 0)),
            pl.BlockSpec((1, 8, _K), lambda b: (b, 0, 0)),
            pl.BlockSpec((1, 1, _K), lambda b: (b, 0, 0)),
        ],
        out_specs=[
            pl.BlockSpec((1, _K, 4), lambda b: (b, 0, 0)),
            pl.BlockSpec((1, 1, _K), lambda b: (b, 0, 0)),
        ],
        out_shape=[
            jax.ShapeDtypeStruct((_B, _K, 4), jnp.float32),
            jax.ShapeDtypeStruct((_B, 1, _K), jnp.float32),
        ],
        scratch_shapes=[pltpu.VMEM((_K, _K), jnp.float32)],
    )(adc, adt, sc3)

    kept_scores = jnp.where(keep[:, 0, :_PRE] > 0.5, top_s, -1.0)
    nms_s, idx = lax.top_k(kept_scores, _POST)
    nms_b = jnp.take_along_axis(boxes_dec[:, :_PRE, :], idx[..., None],
                                axis=1)
    return nms_b, nms_s


# speedup vs baseline: 19.4868x; 1.4180x over previous
import numpy as np

import jax
import jax.numpy as jnp
from jax import lax
from jax.experimental import pallas as pl
from jax.experimental.pallas import tpu as pltpu

_B = 2
_C = 128
_H = 48
_W = 48
_A = 3
_STRIDE = 16
_SIZE = 256.0
_IMG = 768
_PRE = 1000
_POST = 100
_THR = 0.7
_ROWS = _H * _W
_K = 1024
_CLIP = float(np.log(1000.0 / 16.0))


def _anchor_table():
    ratios = np.array([0.5, 1.0, 2.0], np.float32)
    ws = _SIZE * np.sqrt(1.0 / ratios)
    hs = _SIZE * np.sqrt(ratios)
    base = np.stack([-0.5 * (ws - 1), -0.5 * (hs - 1),
                     0.5 * (ws - 1), 0.5 * (hs - 1)], axis=1)
    sx = np.arange(_W, dtype=np.float32) * _STRIDE
    sy = np.arange(_H, dtype=np.float32) * _STRIDE
    gy, gx = np.meshgrid(sy, sx, indexing="ij")
    shifts = np.stack([gx, gy, gx, gy], axis=-1)
    anc = shifts[:, :, None, :] + base[None, None, :, :]
    return jnp.asarray(anc.reshape(-1, 4))


def _conv_head_kernel(x_ref, w1_ref, b1_ref, wh_ref, bh_ref, out_ref):
    xp = x_ref[0]
    x9 = jnp.concatenate(
        [xp[dy:dy + _H, dx:dx + _W, :].reshape(_ROWS, _C)
         for dy in range(3) for dx in range(3)], axis=1)
    acc = lax.dot_general(
        x9, w1_ref[...], (((1,), (0,)), ((), ())),
        preferred_element_type=jnp.float32)
    t = jnp.maximum(acc + b1_ref[0], 0.0)
    h = lax.dot_general(
        t, wh_ref[...], (((1,), (0,)), ((), ())),
        preferred_element_type=jnp.float32) + bh_ref[0]
    col = lax.broadcasted_iota(jnp.int32, h.shape, 1)
    out_ref[0] = jnp.where(col < _A, jax.nn.sigmoid(h), h)


def _decode_from(get):
    wa = get(2) - get(0) + 1.0
    ha = get(3) - get(1) + 1.0
    cxa = get(0) + 0.5 * wa
    cya = get(1) + 0.5 * ha
    dx, dy = get(4), get(5)
    dw = jnp.minimum(get(6), _CLIP)
    dh = jnp.minimum(get(7), _CLIP)
    cx = dx * wa + cxa
    cy = dy * ha + cya
    pw = jnp.exp(dw) * wa
    ph = jnp.exp(dh) * ha
    lim_w = float(_IMG - 1)
    lim_h = float(_IMG - 1)
    x1 = jnp.clip(cx - 0.5 * pw, 0.0, lim_w)
    y1 = jnp.clip(cy - 0.5 * ph, 0.0, lim_h)
    x2 = jnp.clip(cx + 0.5 * pw - 1.0, 0.0, lim_w)
    y2 = jnp.clip(cy + 0.5 * ph - 1.0, 0.0, lim_h)
    return x1, y1, x2, y2


def _nms_kernel(adc_ref, adt_ref, sc_ref, boxes_ref, keep_ref, iou_ref):
    adc = adc_ref[0]
    adt = adt_ref[0]
    x1c, y1c, x2c, y2c = _decode_from(lambda i: adc[:, i:i + 1])
    x1r, y1r, x2r, y2r = _decode_from(lambda i: adt[i:i + 1, :])
    boxes_ref[0] = jnp.concatenate([x1c, y1c, x2c, y2c], axis=1)

    area_c = (x2c - x1c + 1.0) * (y2c - y1c + 1.0)
    area_r = (x2r - x1r + 1.0) * (y2r - y1r + 1.0)
    iw = jnp.maximum(jnp.minimum(x2c, x2r) - jnp.maximum(x1c, x1r) + 1.0, 0.0)
    ih = jnp.maximum(jnp.minimum(y2c, y2r) - jnp.maximum(y1c, y1r) + 1.0, 0.0)
    inter = iw * ih
    iou_ref[...] = inter / (area_c + area_r - inter)

    s = sc_ref[0]
    iota = lax.broadcasted_iota(jnp.int32, (1, _K), 1)

    def body(_, carry):
        active, keep = carry
        cand = jnp.where(active > 0.5, s, -1.0)
        m = jnp.max(cand)
        idx = jnp.min(jnp.where(cand == m, iota, _K))
        valid = m > -0.5
        row = iou_ref[pl.ds(idx, 1), :]
        selmask = ((iota == idx) & valid).astype(jnp.float32)
        keep = keep + selmask
        new_active = active * (row <= _THR).astype(jnp.float32)
        active = jnp.where(valid, new_active, active)
        return active, keep

    active0 = jnp.ones((1, _K), jnp.float32)
    keep0 = jnp.zeros((1, _K), jnp.float32)
    _, keep = lax.fori_loop(0, _POST, body, (active0, keep0))
    keep_ref[0] = keep


def kernel(images, features, W1, b1, W_obj, b_obj, W_box, b_box):
    xp = jnp.pad(jnp.transpose(features, (0, 2, 3, 1)),
                 ((0, 0), (1, 1), (1, 1), (0, 0)))

    w9 = jnp.transpose(W1, (2, 3, 1, 0)).reshape(9 * _C, _C)
    b1r = b1.reshape(1, _C)
    wh = jnp.concatenate([W_obj.reshape(_A, _C).T,
                          W_box.reshape(4 * _A, _C).T,
                          jnp.zeros((_C, 1), jnp.float32)], axis=1)
    bh = jnp.concatenate([b_obj, b_box,
                          jnp.zeros((1,), jnp.float32)]).reshape(1, 16)

    heads = pl.pallas_call(
        _conv_head_kernel,
        grid=(_B,),
        in_specs=[
            pl.BlockSpec((1, 50, 50, _C), lambda b: (b, 0, 0, 0)),
            pl.BlockSpec((9 * _C, _C), lambda b: (0, 0)),
            pl.BlockSpec((1, _C), lambda b: (0, 0)),
            pl.BlockSpec((_C, 16), lambda b: (0, 0)),
            pl.BlockSpec((1, 16), lambda b: (0, 0)),
        ],
        out_specs=pl.BlockSpec((1, _ROWS, 16), lambda b: (b, 0, 0)),
        out_shape=jax.ShapeDtypeStruct((_B, _ROWS, 16), jnp.float32),
    )(xp, w9, b1r, wh, bh)

    scores = heads[..., :_A].reshape(_B, _H * _W * _A)
    deltas = heads[..., _A:_A + 4 * _A].reshape(_B, _H * _W * _A, 4)

    anchors = _anchor_table()
    top_s, top_i = lax.top_k(scores, _PRE)
    anc_t = jnp.take(anchors, top_i, axis=0)
    del_t = jnp.take_along_axis(deltas, top_i[..., None], axis=1)

    adc = jnp.concatenate([anc_t, del_t], axis=-1)
    adc = jnp.pad(adc, ((0, 0), (0, _K - _PRE), (0, 0)))
    adt = jnp.transpose(adc, (0, 2, 1))
    sc3 = jnp.pad(top_s, ((0, 0), (0, _K - _PRE)),
                  constant_values=-1.0).reshape(_B, 1, _K)

    boxes_dec, keep = pl.pallas_call(
        _nms_kernel,
        grid=(_B,),
        in_specs=[
            pl.BlockSpec((1, _K, 8), lambda b: (b, 0, 0)),
            pl.BlockSpec((1, 8, _K), lambda b: (b, 0, 0)),
            pl.BlockSpec((1, 1, _K), lambda b: (b, 0, 0)),
        ],
        out_specs=[
            pl.BlockSpec((1, _K, 4), lambda b: (b, 0, 0)),
            pl.BlockSpec((1, 1, _K), lambda b: (b, 0, 0)),
        ],
        out_shape=[
            jax.ShapeDtypeStruct((_B, _K, 4), jnp.float32),
            jax.ShapeDtypeStruct((_B, 1, _K), jnp.float32),
        ],
        scratch_shapes=[pltpu.VMEM((_K, _K), jnp.float32)],
    )(adc, adt, sc3)

    kept_scores = jnp.where(keep[:, 0, :_PRE] > 0.5, top_s, -1.0)
    nms_s, idx = lax.top_k(kept_scores, _POST)
    nms_b = jnp.take_along_axis(boxes_dec[:, :_PRE, :], idx[..., None],
                                axis=1)
    return nms_b, nms_s
